# baseline (speedup 1.0000x reference)
import jax
import jax.numpy as jnp
from jax.experimental import pallas as pl
from jax.experimental.pallas import tpu as pltpu

_DIM = 1024
_E = 8
_K = 256
_T = 2048
_H = 4096
_HC = 4
_HB = _H // _HC
_JC = 512


def _topk_kernel(lr_ref, ln_ref, z_ref,
                 idx_ref, idxc_ref, idxr_ref, gate_ref):
    f32 = jnp.float32
    i32 = jnp.int32
    noisy = lr_ref[...] + z_ref[...] * jax.nn.softplus(ln_ref[...])
    noisyT = jnp.transpose(noisy)

    bits = jax.lax.bitcast_convert_type(noisyT, i32)
    imin = jnp.int32(-2147483648)
    s = jnp.where(bits < 0, imin - bits, bits)

    kf = jnp.float32(_K)

    def _bisect(_, carry):
        lo, hi = carry
        mid = jnp.right_shift(lo, 1) + jnp.right_shift(hi, 1) + (lo & hi & 1)
        cnt = jnp.sum((s >= mid).astype(f32), axis=1, keepdims=True)
        take = cnt >= kf
        return jnp.where(take, mid, lo), jnp.where(take, hi, mid)

    lo0 = jnp.full((_E, 1), imin, i32)
    hi0 = jnp.full((_E, 1), jnp.int32(2147483647), i32)
    theta, _ = jax.lax.fori_loop(0, 32, _bisect, (lo0, hi0))

    mask_gt = (s > theta).astype(f32)
    mask_eq = (s == theta).astype(f32)
    n_gt = jnp.sum(mask_gt, axis=1, keepdims=True)

    def _prefix(m):
        c = m
        sh = 1
        while sh < _T:
            c = c + jnp.concatenate(
                [jnp.zeros((_E, sh), f32), c[:, :-sh]], axis=1)
            sh *= 2
        return c

    eq_before = _prefix(mask_eq) - mask_eq
    selected = mask_gt + mask_eq * (eq_before < (kf - n_gt)).astype(f32)
    selpos = _prefix(selected) - selected

    j_row = jax.lax.broadcasted_iota(i32, (1, _T), 1).astype(f32)
    p_col = jax.lax.broadcasted_iota(i32, (_K, 1), 0).astype(f32)
    q_row = jax.lax.broadcasted_iota(i32, (1, _K), 1).astype(f32)
    for ee in range(_E):
        selrow = selected[ee:ee + 1, :]
        posrow = selpos[ee:ee + 1, :]
        vrow = noisyT[ee:ee + 1, :]
        ohc = (posrow == p_col) * selrow
        idxc_col = jnp.sum(ohc * j_row, axis=1, keepdims=True)
        valc_col = jnp.sum(ohc * vrow, axis=1, keepdims=True)
        valc_row = jnp.transpose(valc_col)
        beats = (valc_row > valc_col) | ((valc_row == valc_col) &
                                         (q_row < p_col))
        rank = jnp.sum(beats.astype(f32), axis=1, keepdims=True)
        rank_row = jnp.transpose(rank)
        oh2 = (rank_row == p_col).astype(f32)
        idxc_row = jnp.transpose(idxc_col)
        idx_final = jnp.sum(oh2 * idxc_row, axis=1, keepdims=True)
        val_final = jnp.sum(oh2 * valc_row, axis=1, keepdims=True)
        idxc_ref[ee] = idx_final
        idxr_ref[ee] = jnp.transpose(idx_final)
        idx_ref[ee:ee + 1, :] = jnp.transpose(idx_final).astype(jnp.int32)
        m = jnp.max(val_final, axis=0, keepdims=True)
        ex = jnp.exp(val_final - m)
        gate_ref[ee] = ex / jnp.sum(ex, axis=0, keepdims=True)


def _ffn_kernel(xf_ref, idxc_ref, idxr_ref, gate_ref,
                w1_ref, b1_ref, w2_ref, b2_ref,
                out_ref, acc_sc, xbf_sc, xg_sc, yacc_sc):
    e = pl.program_id(0)
    hc = pl.program_id(1)
    f32 = jnp.float32
    bf16 = jnp.bfloat16

    @pl.when(jnp.logical_and(e == 0, hc == 0))
    def _init():
        xf = xf_ref[...]
        xbf_sc[...] = xf.astype(bf16)
        acc_sc[...] = xf

    @pl.when(hc == 0)
    def _gather():
        idx_col = idxc_ref[e]
        j_row = jax.lax.broadcasted_iota(jnp.int32, (1, _T), 1).astype(f32)
        oh = (idx_col == j_row).astype(bf16)
        xg = jax.lax.dot_general(oh, xbf_sc[...], (((1,), (0,)), ((), ())),
                                 preferred_element_type=f32)
        xg_sc[...] = xg.astype(bf16)
        yacc_sc[...] = jnp.broadcast_to(b2_ref[0], (_K, _DIM))

    w1c = w1_ref[0].astype(bf16)
    h = jax.lax.dot_general(xg_sc[...], w1c, (((1,), (0,)), ((), ())),
                            preferred_element_type=f32) + b1_ref[0]
    hb = jnp.maximum(h, 0.0).astype(bf16)
    w2c = w2_ref[0].astype(bf16)
    yacc_sc[...] += jax.lax.dot_general(hb, w2c, (((1,), (0,)), ((), ())),
                                        preferred_element_type=f32)

    @pl.when(hc == _HC - 1)
    def _scatter():
        yg = (yacc_sc[...] * gate_ref[e]).astype(bf16)
        idx_row = idxr_ref[e]
        t_col = jax.lax.broadcasted_iota(jnp.int32, (_T, 1), 0).astype(f32)
        ohT = (t_col == idx_row).astype(bf16)
        acc_sc[...] += jax.lax.dot_general(ohT, yg, (((1,), (0,)), ((), ())),
                                            preferred_element_type=f32)

    @pl.when(jnp.logical_and(e == _E - 1, hc == _HC - 1))
    def _writeout():
        out_ref[...] = acc_sc[...]


def _topk_call(lr, ln, z_te, interpret=False):
    return pl.pallas_call(
        _topk_kernel,
        out_shape=[
            jax.ShapeDtypeStruct((_E, _K), jnp.int32),
            jax.ShapeDtypeStruct((_E, _K, 1), jnp.float32),
            jax.ShapeDtypeStruct((_E, 1, _K), jnp.float32),
            jax.ShapeDtypeStruct((_E, _K, 1), jnp.float32),
        ],
        interpret=interpret,
    )(lr, ln, z_te)


def _ffn_call(xf, idxc, idxr, gates, W1, b1, W2, b2, interpret=False):
    return pl.pallas_call(
        _ffn_kernel,
        grid=(_E, _HC),
        in_specs=[
            pl.BlockSpec((_T, _DIM), lambda e, hc: (0, 0)),
            pl.BlockSpec((_E, _K, 1), lambda e, hc: (0, 0, 0)),
            pl.BlockSpec((_E, 1, _K), lambda e, hc: (0, 0, 0)),
            pl.BlockSpec((_E, _K, 1), lambda e, hc: (0, 0, 0)),
            pl.BlockSpec((1, _DIM, _HB), lambda e, hc: (e, 0, hc)),
            pl.BlockSpec((1, 1, _HB), lambda e, hc: (e, 0, hc)),
            pl.BlockSpec((1, _HB, _DIM), lambda e, hc: (e, hc, 0)),
            pl.BlockSpec((1, 1, _DIM), lambda e, hc: (e, 0, 0)),
        ],
        out_specs=pl.BlockSpec((_T, _DIM), lambda e, hc: (0, 0)),
        out_shape=jax.ShapeDtypeStruct((_T, _DIM), jnp.float32),
        scratch_shapes=[
            pltpu.VMEM((_T, _DIM), jnp.float32),
            pltpu.VMEM((_T, _DIM), jnp.bfloat16),
            pltpu.VMEM((_K, _DIM), jnp.bfloat16),
            pltpu.VMEM((_K, _DIM), jnp.float32),
        ],
        compiler_params=pltpu.CompilerParams(
            dimension_semantics=("arbitrary", "arbitrary")),
        interpret=interpret,
    )(xf, idxc, idxr, gates, W1, b1.reshape(_E, 1, _H), W2,
      b2.reshape(_E, 1, _DIM))


def kernel(x, Wr, br, Wn, bn, W1, b1, W2, b2, interpret=False):
    bs, seq, dim = x.shape
    xf = x.reshape(seq, dim)
    lr = (x @ Wr + br).reshape(-1, _E)
    ln = (x @ Wn + bn).reshape(-1, _E)
    z_te = jnp.transpose(
        jax.random.normal(jax.random.key(42), (_E, seq), dtype=jnp.float32))
    idx, idxc, idxr, gates = _topk_call(lr, ln, z_te, interpret=interpret)
    out = _ffn_call(xf, idxc, idxr, gates,
                    W1, b1, W2, b2, interpret=interpret)
    return out.reshape(bs, seq, dim), idx


# speedup vs baseline: 1.0475x; 1.0475x over previous
import jax
import jax.numpy as jnp
from jax.experimental import pallas as pl
from jax.experimental.pallas import tpu as pltpu

_DIM = 1024
_E = 8
_K = 256
_T = 2048
_H = 4096
_HC = 2
_HB = _H // _HC


def _topk_kernel(lr_ref, ln_ref, z_ref,
                 idx_ref, idxc_ref, idxr_ref, gate_ref):
    f32 = jnp.float32
    i32 = jnp.int32
    noisy = lr_ref[...] + z_ref[...] * jax.nn.softplus(ln_ref[...])
    noisyT = jnp.transpose(noisy)

    bits = jax.lax.bitcast_convert_type(noisyT, i32)
    imin = jnp.int32(-2147483648)
    s = jnp.where(bits < 0, imin - bits, bits)

    kf = jnp.float32(_K)

    def _bisect(_, carry):
        lo, hi = carry
        mid = jnp.right_shift(lo, 1) + jnp.right_shift(hi, 1) + (lo & hi & 1)
        cnt = jnp.sum((s >= mid).astype(f32), axis=1, keepdims=True)
        take = cnt >= kf
        return jnp.where(take, mid, lo), jnp.where(take, hi, mid)

    lo0 = jnp.full((_E, 1), imin, i32)
    hi0 = jnp.full((_E, 1), jnp.int32(2147483647), i32)
    theta, _ = jax.lax.fori_loop(0, 32, _bisect, (lo0, hi0))

    mask_gt = (s > theta).astype(f32)
    mask_eq = (s == theta).astype(f32)
    n_gt = jnp.sum(mask_gt, axis=1, keepdims=True)

    def _prefix(m):
        c = m
        sh = 1
        while sh < _T:
            c = c + jnp.concatenate(
                [jnp.zeros((_E, sh), f32), c[:, :-sh]], axis=1)
            sh *= 2
        return c

    eq_before = _prefix(mask_eq) - mask_eq
    selected = mask_gt + mask_eq * (eq_before < (kf - n_gt)).astype(f32)
    selpos = _prefix(selected) - selected

    j_row = jax.lax.broadcasted_iota(i32, (1, _T), 1).astype(f32)
    p_col = jax.lax.broadcasted_iota(i32, (_K, 1), 0).astype(f32)
    q_row = jax.lax.broadcasted_iota(i32, (1, _K), 1).astype(f32)
    for ee in range(_E):
        selrow = selected[ee:ee + 1, :]
        posrow = selpos[ee:ee + 1, :]
        vrow = noisyT[ee:ee + 1, :]
        ohc = (posrow == p_col) * selrow
        idxc_col = jnp.sum(ohc * j_row, axis=1, keepdims=True)
        valc_col = jnp.sum(ohc * vrow, axis=1, keepdims=True)
        valc_row = jnp.transpose(valc_col)
        beats = (valc_row > valc_col) | ((valc_row == valc_col) &
                                         (q_row < p_col))
        rank = jnp.sum(beats.astype(f32), axis=1, keepdims=True)
        rank_row = jnp.transpose(rank)
        oh2 = (rank_row == p_col).astype(f32)
        idxc_row = jnp.transpose(idxc_col)
        idx_final = jnp.sum(oh2 * idxc_row, axis=1, keepdims=True)
        val_final = jnp.sum(oh2 * valc_row, axis=1, keepdims=True)
        idxc_ref[ee] = idx_final
        idxr_ref[ee] = jnp.transpose(idx_final)
        idx_ref[ee:ee + 1, :] = jnp.transpose(idx_final).astype(jnp.int32)
        m = jnp.max(val_final, axis=0, keepdims=True)
        ex = jnp.exp(val_final - m)
        gate_ref[ee] = ex / jnp.sum(ex, axis=0, keepdims=True)


def _ffn_kernel(xf_ref, idxc_ref, idxr_ref, gate_ref,
                w1_ref, b1_ref, w2_ref, b2_ref,
                acc_ref, xbf_sc, xg_sc, yacc_sc):
    e = pl.program_id(0)
    hc = pl.program_id(1)
    f32 = jnp.float32
    bf16 = jnp.bfloat16

    @pl.when(jnp.logical_and(e == 0, hc == 0))
    def _init():
        xf = xf_ref[...]
        xbf_sc[...] = xf.astype(bf16)
        acc_ref[...] = xf

    @pl.when(hc == 0)
    def _gather():
        idx_col = idxc_ref[e]
        j_row = jax.lax.broadcasted_iota(jnp.int32, (1, _T), 1).astype(f32)
        oh = (idx_col == j_row).astype(bf16)
        xg = jax.lax.dot_general(oh, xbf_sc[...], (((1,), (0,)), ((), ())),
                                 preferred_element_type=f32)
        xg_sc[...] = xg.astype(bf16)
        yacc_sc[...] = jnp.broadcast_to(b2_ref[0], (_K, _DIM))

    w1c = w1_ref[0].astype(bf16)
    h = jax.lax.dot_general(xg_sc[...], w1c, (((1,), (0,)), ((), ())),
                            preferred_element_type=f32) + b1_ref[0]
    hb = jnp.maximum(h, 0.0).astype(bf16)
    w2c = w2_ref[0].astype(bf16)
    yacc_sc[...] += jax.lax.dot_general(hb, w2c, (((1,), (0,)), ((), ())),
                                        preferred_element_type=f32)

    @pl.when(hc == _HC - 1)
    def _scatter():
        yg = (yacc_sc[...] * gate_ref[e]).astype(bf16)
        idx_row = idxr_ref[e]
        t_col = jax.lax.broadcasted_iota(jnp.int32, (_T, 1), 0).astype(f32)
        ohT = (t_col == idx_row).astype(bf16)
        acc_ref[...] += jax.lax.dot_general(ohT, yg, (((1,), (0,)), ((), ())),
                                            preferred_element_type=f32)


def _topk_call(lr, ln, z_te, interpret=False):
    return pl.pallas_call(
        _topk_kernel,
        out_shape=[
            jax.ShapeDtypeStruct((_E, _K), jnp.int32),
            jax.ShapeDtypeStruct((_E, _K, 1), jnp.float32),
            jax.ShapeDtypeStruct((_E, 1, _K), jnp.float32),
            jax.ShapeDtypeStruct((_E, _K, 1), jnp.float32),
        ],
        interpret=interpret,
    )(lr, ln, z_te)


def _ffn_call(xf, idxc, idxr, gates, W1, b1, W2, b2, interpret=False):
    return pl.pallas_call(
        _ffn_kernel,
        grid=(_E, _HC),
        in_specs=[
            pl.BlockSpec((_T, _DIM), lambda e, hc: (0, 0)),
            pl.BlockSpec((_E, _K, 1), lambda e, hc: (0, 0, 0)),
            pl.BlockSpec((_E, 1, _K), lambda e, hc: (0, 0, 0)),
            pl.BlockSpec((_E, _K, 1), lambda e, hc: (0, 0, 0)),
            pl.BlockSpec((1, _DIM, _HB), lambda e, hc: (e, 0, hc)),
            pl.BlockSpec((1, 1, _HB), lambda e, hc: (e, 0, hc)),
            pl.BlockSpec((1, _HB, _DIM), lambda e, hc: (e, hc, 0)),
            pl.BlockSpec((1, 1, _DIM), lambda e, hc: (e, 0, 0)),
        ],
        out_specs=pl.BlockSpec((_T, _DIM), lambda e, hc: (0, 0)),
        out_shape=jax.ShapeDtypeStruct((_T, _DIM), jnp.float32),
        scratch_shapes=[
            pltpu.VMEM((_T, _DIM), jnp.bfloat16),
            pltpu.VMEM((_K, _DIM), jnp.bfloat16),
            pltpu.VMEM((_K, _DIM), jnp.float32),
        ],
        compiler_params=pltpu.CompilerParams(
            dimension_semantics=("arbitrary", "arbitrary")),
        interpret=interpret,
    )(xf, idxc, idxr, gates, W1, b1.reshape(_E, 1, _H), W2,
      b2.reshape(_E, 1, _DIM))


def kernel(x, Wr, br, Wn, bn, W1, b1, W2, b2):
    bs, seq, dim = x.shape
    xf = x.reshape(seq, dim)
    lr = (x @ Wr + br).reshape(-1, _E)
    ln = (x @ Wn + bn).reshape(-1, _E)
    z_te = jnp.transpose(
        jax.random.normal(jax.random.key(42), (_E, seq), dtype=jnp.float32))
    idx, idxc, idxr, gates = _topk_call(lr, ln, z_te)
    out = _ffn_call(xf, idxc, idxr, gates, W1, b1, W2, b2)
    return out.reshape(bs, seq, dim), idx
